# Initial kernel scaffold; baseline (speedup 1.0000x reference)
import jax
import jax.numpy as jnp
from jax.experimental import pallas as pl


def kernel(frameIndices, numFrames, frameEmbed_weight):
    raise NotImplementedError("write your pallas kernel here")


# speedup vs baseline: 2.5601x; 2.5601x over previous
import jax
import jax.numpy as jnp
from jax import lax
from jax.experimental import pallas as pl
from jax.experimental.pallas import tpu as pltpu
from jax.experimental.pallas import tpu_sc as plsc

NC = 2
NS = 16
NW = NC * NS

B = 4096 * 200
D = 256
B_PER_W = B // NW
CHUNK = 128
N_CHUNKS = B_PER_W // CHUNK


def _gather_body(idx_hbm, table_hbm, out_hbm, idx_v, rows_v, sem):
    wid = lax.axis_index("s") * NC + lax.axis_index("c")
    base = wid * B_PER_W

    def chunk(k, carry):
        off = base + k * CHUNK
        pltpu.sync_copy(idx_hbm.at[pl.ds(off, CHUNK)], idx_v)
        pltpu.async_copy(table_hbm.at[idx_v], rows_v, sem).wait()
        pltpu.sync_copy(rows_v, out_hbm.at[pl.ds(off, CHUNK)])
        return carry

    lax.fori_loop(0, N_CHUNKS, chunk, 0)


def kernel(frameIndices, numFrames, frameEmbed_weight):
    del numFrames
    idx = frameIndices.astype(jnp.int32).reshape(B)
    mesh = plsc.VectorSubcoreMesh(
        core_axis_name="c", subcore_axis_name="s", num_cores=NC, num_subcores=NS
    )
    out = pl.kernel(
        _gather_body,
        out_type=jax.ShapeDtypeStruct((B, D), jnp.float32),
        mesh=mesh,
        scratch_types=[
            pltpu.VMEM((CHUNK,), jnp.int32),
            pltpu.VMEM((CHUNK, D), jnp.float32),
            pltpu.SemaphoreType.DMA,
        ],
    )(idx, frameEmbed_weight)
    return out.reshape(frameIndices.shape[0], frameIndices.shape[1], D)


# speedup vs baseline: 2.5924x; 1.0126x over previous
import jax
import jax.numpy as jnp
from jax import lax
from jax.experimental import pallas as pl
from jax.experimental.pallas import tpu as pltpu
from jax.experimental.pallas import tpu_sc as plsc

NC = 2
NS = 16
NW = NC * NS

B = 4096 * 200
D = 256
B_PER_W = B // NW
CHUNK = 128
N_CHUNKS = B_PER_W // CHUNK


def _gather_body(idx_hbm, table_hbm, out_hbm, idx_v, rows0, rows1, gsem, osem0, osem1):
    wid = lax.axis_index("s") * NC + lax.axis_index("c")
    base = wid * B_PER_W
    pltpu.sync_copy(idx_hbm.at[wid], idx_v)
    rows = (rows0, rows1)
    osem = (osem0, osem1)

    def pair(kk, carry):
        for b in range(2):
            k = 2 * kk + b
            off = base + k * CHUNK

            @pl.when(kk > 0)
            def _wait_prev():
                pltpu.make_async_copy(
                    rows[b], out_hbm.at[pl.ds(off - 2 * CHUNK, CHUNK)], osem[b]
                ).wait()

            pltpu.async_copy(table_hbm.at[idx_v.at[k]], rows[b], gsem).wait()
            pltpu.async_copy(rows[b], out_hbm.at[pl.ds(off, CHUNK)], osem[b])
        return carry

    lax.fori_loop(0, N_CHUNKS // 2, pair, 0)
    end = base + N_CHUNKS * CHUNK
    for b in range(2):
        pltpu.make_async_copy(
            rows[b], out_hbm.at[pl.ds(end - (2 - b) * CHUNK, CHUNK)], osem[b]
        ).wait()


def kernel(frameIndices, numFrames, frameEmbed_weight):
    del numFrames
    idx = frameIndices.astype(jnp.int32).reshape(NW, N_CHUNKS, CHUNK)
    mesh = plsc.VectorSubcoreMesh(
        core_axis_name="c", subcore_axis_name="s", num_cores=NC, num_subcores=NS
    )
    out = pl.kernel(
        _gather_body,
        out_type=jax.ShapeDtypeStruct((B, D), jnp.float32),
        mesh=mesh,
        scratch_types=[
            pltpu.VMEM((N_CHUNKS, CHUNK), jnp.int32),
            pltpu.VMEM((CHUNK, D), jnp.float32),
            pltpu.VMEM((CHUNK, D), jnp.float32),
            pltpu.SemaphoreType.DMA,
            pltpu.SemaphoreType.DMA,
            pltpu.SemaphoreType.DMA,
        ],
    )(idx, frameEmbed_weight)
    return out.reshape(frameIndices.shape[0], frameIndices.shape[1], D)
